# Initial kernel scaffold; baseline (speedup 1.0000x reference)
import jax
import jax.numpy as jnp
from jax.experimental import pallas as pl


def kernel(src, tgt, val, hyper, uEmbeds, iEmbeds, edgeTrans):
    raise NotImplementedError("write your pallas kernel here")


# speedup vs baseline: 3.4176x; 3.4176x over previous
import jax
import jax.numpy as jnp
from jax import lax
from jax.experimental import pallas as pl
from jax.experimental.pallas import tpu as pltpu
from jax.experimental.pallas import tpu_sc as plsc

USER = 5000
ITEM = 5000
LATDIM = 32
ETYPE = 4
HYPERNUM = 64
E = 160000

NC = 2
NS = 16
NW = NC * NS
CH = 128
NCHUNKS = E // CH
NCH_BASE = NCHUNKS // NW
EXTRA = NCHUNKS % NW

TBL_N = 5120
STG = TBL_N // NS
W = 4 * LATDIM

ROWS_PT = 320
ACC_N = NS * ROWS_PT

BE = 1280
NB = E // BE


def _leaky(x):
    return jnp.where(x >= 0, x, 0.5 * x)


CH1 = 64
NCHUNKS1 = E // CH1
NCH1_BASE = NCHUNKS1 // NW
EXTRA1 = NCHUNKS1 % NW


def _gather_body(ue_hbm, ie_hbm, src_hbm, tgt_hbm, srows_hbm, trows_hbm,
                 usp, isp, sidxv, tidxv, urows, irows, u32, i32,
                 sem_u, sem_i):
    c = lax.axis_index("c")
    s = lax.axis_index("s")
    wid = s * NC + c
    nch = jnp.where(wid < EXTRA1, NCH1_BASE + 1, NCH1_BASE)

    for k in range(STG // CH1):
        off = s * STG + k * CH1
        pltpu.sync_copy(ue_hbm.at[pl.ds(off, CH1)], urows)
        pltpu.sync_copy(urows, usp.at[pl.ds(off, CH1)])
        pltpu.sync_copy(ie_hbm.at[pl.ds(off, CH1)], irows)
        pltpu.sync_copy(irows, isp.at[pl.ds(off, CH1)])
    plsc.subcore_barrier()

    def chunk(j, carry):
        base = (j * NW + wid) * CH1
        pltpu.sync_copy(src_hbm.at[pl.ds(base, CH1)], sidxv)
        pltpu.sync_copy(tgt_hbm.at[pl.ds(base, CH1)], tidxv)
        cp_u = pltpu.async_copy(usp.at[sidxv], urows, sem_u)
        cp_i = pltpu.async_copy(isp.at[tidxv], irows, sem_i)
        cp_u.wait()
        cp_i.wait()

        def cprow(r, cc):
            u32[r, pl.ds(0, 16)] = urows[r, pl.ds(0, 16)]
            u32[r, pl.ds(16, 16)] = urows[r, pl.ds(16, 16)]
            i32[r, pl.ds(0, 16)] = irows[r, pl.ds(0, 16)]
            i32[r, pl.ds(16, 16)] = irows[r, pl.ds(16, 16)]
            return cc

        lax.fori_loop(0, CH1, cprow, 0)
        pltpu.sync_copy(u32, srows_hbm.at[pl.ds(base, CH1)])
        pltpu.sync_copy(i32, trows_hbm.at[pl.ds(base, CH1)])
        return carry

    lax.fori_loop(0, nch, chunk, 0)


def _gather(uE4, iE4, src, tgt):
    mesh = plsc.VectorSubcoreMesh(core_axis_name="c", subcore_axis_name="s")
    return pl.kernel(
        _gather_body,
        out_type=(
            jax.ShapeDtypeStruct((E, LATDIM), jnp.float32),
            jax.ShapeDtypeStruct((E, LATDIM), jnp.float32),
        ),
        mesh=mesh,
        scratch_types=[
            pltpu.VMEM_SHARED((TBL_N, W), jnp.float32),
            pltpu.VMEM_SHARED((TBL_N, W), jnp.float32),
            pltpu.VMEM((CH1,), jnp.int32),
            pltpu.VMEM((CH1,), jnp.int32),
            pltpu.VMEM((CH1, W), jnp.float32),
            pltpu.VMEM((CH1, W), jnp.float32),
            pltpu.VMEM((CH1, LATDIM), jnp.float32),
            pltpu.VMEM((CH1, LATDIM), jnp.float32),
            pltpu.SemaphoreType.DMA,
            pltpu.SemaphoreType.DMA,
        ],
    )(uE4, iE4, src, tgt)


def _dense_body(s_ref, t_ref, v_ref, et_ref, h_ref, out_ref, Xs, A, P, xt):
    p = pl.program_id(0)
    i = pl.program_id(1)

    @pl.when(i == 0)
    def _():
        P[...] = _leaky(A[...])
        A[...] = jnp.zeros((LATDIM, HYPERNUM), jnp.float32)

    h = h_ref[...]

    @pl.when(p == 0)
    def _():
        sb = s_ref[...]
        vb = v_ref[0, 0, :].astype(jnp.int32)
        x0 = jnp.zeros((BE, LATDIM), jnp.float32)
        for v in range(ETYPE):
            y = jnp.dot(sb, et_ref[v], preferred_element_type=jnp.float32)
            mask = (vb == v).astype(jnp.float32)[:, None]
            x0 = x0 + mask * y
        x0 = x0 * t_ref[...]
        xt[...] = x0.T

    @pl.when(p > 0)
    def _():
        hp = lax.dot_general(P[...], h, (((1,), (1,)), ((), ())),
                             preferred_element_type=jnp.float32)
        xt[...] = _leaky(hp) + Xs[:, pl.ds(i * BE, BE)]

    x = xt[...]

    @pl.when(p < 2)
    def _():
        Xs[:, pl.ds(i * BE, BE)] = x
        A[...] += lax.dot_general(x, h, (((1,), (0,)), ((), ())),
                                  preferred_element_type=jnp.float32)

    @pl.when(p == 2)
    def _():
        out_ref[...] = x.T


def _dense(srows, trows, val3, edgeTrans, hyper):
    return pl.pallas_call(
        _dense_body,
        grid=(3, NB),
        in_specs=[
            pl.BlockSpec((BE, LATDIM), lambda p, i: (jnp.where(p == 0, i, 0), 0)),
            pl.BlockSpec((BE, LATDIM), lambda p, i: (jnp.where(p == 0, i, 0), 0)),
            pl.BlockSpec((1, 1, BE), lambda p, i: (jnp.where(p == 0, i, 0), 0, 0)),
            pl.BlockSpec((ETYPE, LATDIM, LATDIM), lambda p, i: (0, 0, 0)),
            pl.BlockSpec((BE, HYPERNUM), lambda p, i: (i, 0)),
        ],
        out_specs=pl.BlockSpec((BE, LATDIM), lambda p, i: (jnp.where(p == 2, i, 0), 0)),
        out_shape=jax.ShapeDtypeStruct((E, LATDIM), jnp.float32),
        scratch_shapes=[
            pltpu.VMEM((LATDIM, E), jnp.float32),
            pltpu.VMEM((LATDIM, HYPERNUM), jnp.float32),
            pltpu.VMEM((LATDIM, HYPERNUM), jnp.float32),
            pltpu.VMEM((LATDIM, BE), jnp.float32),
        ],
        compiler_params=pltpu.CompilerParams(
            dimension_semantics=("arbitrary", "arbitrary")),
    )(srows, trows, val3, edgeTrans, hyper)


def _scatter_body(x2_hbm, src_hbm, tgt_hbm, sp_hbm, tp_hbm,
                  rows, r32, sidxv, tidxv, accS, accT):
    c = lax.axis_index("c")
    s = lax.axis_index("s")
    wid = s * NC + c
    nch = jnp.where(wid < EXTRA, NCH_BASE + 1, NCH_BASE)

    def zrow(r, cc):
        for q in range(W // 16):
            rows[r, pl.ds(q * 16, 16)] = jnp.zeros((16,), jnp.float32)
        return cc

    lax.fori_loop(0, CH, zrow, 0)
    for acc in (accS, accT):
        pltpu.sync_copy(rows, acc.at[pl.ds(s * ROWS_PT, CH)])
        pltpu.sync_copy(rows, acc.at[pl.ds(s * ROWS_PT + CH, CH)])
        pltpu.sync_copy(rows.at[pl.ds(0, ROWS_PT - 2 * CH)],
                        acc.at[pl.ds(s * ROWS_PT + 2 * CH, ROWS_PT - 2 * CH)])
    plsc.subcore_barrier()

    def chunk(j, cc):
        base = (j * NW + wid) * CH
        pltpu.sync_copy(x2_hbm.at[pl.ds(base, CH)], r32)
        pltpu.sync_copy(src_hbm.at[pl.ds(base, CH)], sidxv)
        pltpu.sync_copy(tgt_hbm.at[pl.ds(base, CH)], tidxv)

        def cprow(r, cc2):
            rows[r, pl.ds(0, 16)] = r32[r, pl.ds(0, 16)]
            rows[r, pl.ds(16, 16)] = r32[r, pl.ds(16, 16)]
            return cc2

        lax.fori_loop(0, CH, cprow, 0)
        pltpu.sync_copy(rows, accS.at[sidxv], add=True)
        pltpu.sync_copy(rows, accT.at[tidxv], add=True)
        return cc

    lax.fori_loop(0, nch, chunk, 0)
    plsc.subcore_barrier()
    tsl = pl.ds(s * ROWS_PT, ROWS_PT)
    pltpu.sync_copy(accS.at[tsl], sp_hbm.at[c, tsl])
    pltpu.sync_copy(accT.at[tsl], tp_hbm.at[c, tsl])


def _scatter(x2, src, tgt):
    mesh = plsc.VectorSubcoreMesh(core_axis_name="c", subcore_axis_name="s")
    return pl.kernel(
        _scatter_body,
        out_type=(
            jax.ShapeDtypeStruct((NC, ACC_N, W), jnp.float32),
            jax.ShapeDtypeStruct((NC, ACC_N, W), jnp.float32),
        ),
        mesh=mesh,
        scratch_types=[
            pltpu.VMEM((CH, W), jnp.float32),
            pltpu.VMEM((CH, LATDIM), jnp.float32),
            pltpu.VMEM((CH,), jnp.int32),
            pltpu.VMEM((CH,), jnp.int32),
            pltpu.VMEM_SHARED((ACC_N, W), jnp.float32),
            pltpu.VMEM_SHARED((ACC_N, W), jnp.float32),
        ],
    )(x2, src, tgt)


def kernel(src, tgt, val, hyper, uEmbeds, iEmbeds, edgeTrans):
    uEp = jnp.zeros((TBL_N, LATDIM), jnp.float32).at[:USER].set(uEmbeds)
    iEp = jnp.zeros((TBL_N, LATDIM), jnp.float32).at[:ITEM].set(iEmbeds)
    uE4 = jnp.concatenate([uEp] * 4, axis=1)
    iE4 = jnp.concatenate([iEp] * 4, axis=1)
    srows, trows = _gather(uE4, iE4, src, tgt)
    val3 = val.reshape(NB, 1, BE)
    x2 = _dense(srows, trows, val3, edgeTrans, hyper)
    sp, tp = _scatter(x2, src, tgt)
    srcOut = (sp[0] + sp[1])[:USER, :LATDIM]
    tgtOut = (tp[0] + tp[1])[:ITEM, :LATDIM]
    return (srcOut, tgtOut)


# speedup vs baseline: 3.6666x; 1.0728x over previous
import jax
import jax.numpy as jnp
from jax import lax
from jax.experimental import pallas as pl
from jax.experimental.pallas import tpu as pltpu
from jax.experimental.pallas import tpu_sc as plsc

USER = 5000
ITEM = 5000
LATDIM = 32
ETYPE = 4
HYPERNUM = 64
E = 160000

NC = 2
NS = 16
NW = NC * NS
CH = 128
NCHUNKS = E // CH
NCH_BASE = NCHUNKS // NW
EXTRA = NCHUNKS % NW

TBL_N = 5120
STG = TBL_N // NS
W = 4 * LATDIM

ROWS_PT = 320
ACC_N = NS * ROWS_PT

BE = 1280
NB = E // BE


def _leaky(x):
    return jnp.where(x >= 0, x, 0.5 * x)


CH1 = 64
NCHUNKS1 = E // CH1
NCH1_BASE = NCHUNKS1 // NW
EXTRA1 = NCHUNKS1 % NW


def _gather_body(ue_hbm, ie_hbm, src_hbm, tgt_hbm, srows_hbm, trows_hbm,
                 usp, isp, sidxv, tidxv, urows, irows, u32, i32,
                 sem_u, sem_i, sem_o1, sem_o2):
    c = lax.axis_index("c")
    s = lax.axis_index("s")
    wid = s * NC + c
    nch = jnp.where(wid < EXTRA1, NCH1_BASE + 1, NCH1_BASE)

    for k in range(STG // CH1):
        off = s * STG + k * CH1
        pltpu.sync_copy(ue_hbm.at[pl.ds(off, CH1)], urows)
        pltpu.sync_copy(urows, usp.at[pl.ds(off, CH1)])
        pltpu.sync_copy(ie_hbm.at[pl.ds(off, CH1)], irows)
        pltpu.sync_copy(irows, isp.at[pl.ds(off, CH1)])
    plsc.subcore_barrier()

    def chunk(j, carry):
        base = (j * NW + wid) * CH1
        pltpu.sync_copy(src_hbm.at[pl.ds(base, CH1)], sidxv)
        pltpu.sync_copy(tgt_hbm.at[pl.ds(base, CH1)], tidxv)
        cp_u = pltpu.async_copy(usp.at[sidxv], urows, sem_u)
        cp_i = pltpu.async_copy(isp.at[tidxv], irows, sem_i)
        cp_u.wait()
        cp_i.wait()

        @pl.when(j > 0)
        def _():
            pbase = ((j - 1) * NW + wid) * CH1
            pltpu.make_async_copy(u32, srows_hbm.at[pl.ds(pbase, CH1)],
                                  sem_o1).wait()
            pltpu.make_async_copy(i32, trows_hbm.at[pl.ds(pbase, CH1)],
                                  sem_o2).wait()

        def cprow(r4, cc):
            for d in range(4):
                r = r4 * 4 + d
                u32[r, pl.ds(0, 16)] = urows[r, pl.ds(0, 16)]
                u32[r, pl.ds(16, 16)] = urows[r, pl.ds(16, 16)]
                i32[r, pl.ds(0, 16)] = irows[r, pl.ds(0, 16)]
                i32[r, pl.ds(16, 16)] = irows[r, pl.ds(16, 16)]
            return cc

        lax.fori_loop(0, CH1 // 4, cprow, 0)
        pltpu.async_copy(u32, srows_hbm.at[pl.ds(base, CH1)], sem_o1)
        pltpu.async_copy(i32, trows_hbm.at[pl.ds(base, CH1)], sem_o2)
        return carry

    lax.fori_loop(0, nch, chunk, 0)
    fbase = ((nch - 1) * NW + wid) * CH1
    pltpu.make_async_copy(u32, srows_hbm.at[pl.ds(fbase, CH1)], sem_o1).wait()
    pltpu.make_async_copy(i32, trows_hbm.at[pl.ds(fbase, CH1)], sem_o2).wait()


def _gather(uE4, iE4, src, tgt):
    mesh = plsc.VectorSubcoreMesh(core_axis_name="c", subcore_axis_name="s")
    return pl.kernel(
        _gather_body,
        out_type=(
            jax.ShapeDtypeStruct((E, LATDIM), jnp.float32),
            jax.ShapeDtypeStruct((E, LATDIM), jnp.float32),
        ),
        mesh=mesh,
        scratch_types=[
            pltpu.VMEM_SHARED((TBL_N, W), jnp.float32),
            pltpu.VMEM_SHARED((TBL_N, W), jnp.float32),
            pltpu.VMEM((CH1,), jnp.int32),
            pltpu.VMEM((CH1,), jnp.int32),
            pltpu.VMEM((CH1, W), jnp.float32),
            pltpu.VMEM((CH1, W), jnp.float32),
            pltpu.VMEM((CH1, LATDIM), jnp.float32),
            pltpu.VMEM((CH1, LATDIM), jnp.float32),
            pltpu.SemaphoreType.DMA,
            pltpu.SemaphoreType.DMA,
            pltpu.SemaphoreType.DMA,
            pltpu.SemaphoreType.DMA,
        ],
    )(uE4, iE4, src, tgt)


def _dense_body(s_ref, t_ref, v_ref, et_ref, h_ref, out_ref, Xs, A, P, xt):
    p = pl.program_id(0)
    i = pl.program_id(1)

    @pl.when(i == 0)
    def _():
        P[...] = _leaky(A[...])
        A[...] = jnp.zeros((LATDIM, HYPERNUM), jnp.float32)

    h = h_ref[...]

    @pl.when(p == 0)
    def _():
        sb = s_ref[...]
        vb = v_ref[0, 0, :].astype(jnp.int32)
        x0 = jnp.zeros((BE, LATDIM), jnp.float32)
        for v in range(ETYPE):
            y = jnp.dot(sb, et_ref[v], preferred_element_type=jnp.float32)
            mask = (vb == v).astype(jnp.float32)[:, None]
            x0 = x0 + mask * y
        x0 = x0 * t_ref[...]
        xt[...] = x0.T

    @pl.when(p > 0)
    def _():
        hp = lax.dot_general(P[...], h, (((1,), (1,)), ((), ())),
                             preferred_element_type=jnp.float32)
        xt[...] = _leaky(hp) + Xs[:, pl.ds(i * BE, BE)]

    x = xt[...]

    @pl.when(p < 2)
    def _():
        Xs[:, pl.ds(i * BE, BE)] = x
        A[...] += lax.dot_general(x, h, (((1,), (0,)), ((), ())),
                                  preferred_element_type=jnp.float32)

    @pl.when(p == 2)
    def _():
        out_ref[...] = x.T


def _dense(srows, trows, val3, edgeTrans, hyper):
    return pl.pallas_call(
        _dense_body,
        grid=(3, NB),
        in_specs=[
            pl.BlockSpec((BE, LATDIM), lambda p, i: (jnp.where(p == 0, i, 0), 0)),
            pl.BlockSpec((BE, LATDIM), lambda p, i: (jnp.where(p == 0, i, 0), 0)),
            pl.BlockSpec((1, 1, BE), lambda p, i: (jnp.where(p == 0, i, 0), 0, 0)),
            pl.BlockSpec((ETYPE, LATDIM, LATDIM), lambda p, i: (0, 0, 0)),
            pl.BlockSpec((BE, HYPERNUM), lambda p, i: (i, 0)),
        ],
        out_specs=pl.BlockSpec((BE, LATDIM), lambda p, i: (jnp.where(p == 2, i, 0), 0)),
        out_shape=jax.ShapeDtypeStruct((E, LATDIM), jnp.float32),
        scratch_shapes=[
            pltpu.VMEM((LATDIM, E), jnp.float32),
            pltpu.VMEM((LATDIM, HYPERNUM), jnp.float32),
            pltpu.VMEM((LATDIM, HYPERNUM), jnp.float32),
            pltpu.VMEM((LATDIM, BE), jnp.float32),
        ],
        compiler_params=pltpu.CompilerParams(
            dimension_semantics=("arbitrary", "arbitrary")),
    )(srows, trows, val3, edgeTrans, hyper)


def _scatter_body(x2_hbm, src_hbm, tgt_hbm, sp_hbm, tp_hbm,
                  rows, r32, sidxv, tidxv, accS, accT):
    c = lax.axis_index("c")
    s = lax.axis_index("s")
    wid = s * NC + c
    nch = jnp.where(wid < EXTRA, NCH_BASE + 1, NCH_BASE)

    def zrow(r, cc):
        for q in range(W // 16):
            rows[r, pl.ds(q * 16, 16)] = jnp.zeros((16,), jnp.float32)
        return cc

    lax.fori_loop(0, CH, zrow, 0)
    for acc in (accS, accT):
        pltpu.sync_copy(rows, acc.at[pl.ds(s * ROWS_PT, CH)])
        pltpu.sync_copy(rows, acc.at[pl.ds(s * ROWS_PT + CH, CH)])
        pltpu.sync_copy(rows.at[pl.ds(0, ROWS_PT - 2 * CH)],
                        acc.at[pl.ds(s * ROWS_PT + 2 * CH, ROWS_PT - 2 * CH)])
    plsc.subcore_barrier()

    def chunk(j, cc):
        base = (j * NW + wid) * CH
        pltpu.sync_copy(x2_hbm.at[pl.ds(base, CH)], r32)
        pltpu.sync_copy(src_hbm.at[pl.ds(base, CH)], sidxv)
        pltpu.sync_copy(tgt_hbm.at[pl.ds(base, CH)], tidxv)

        def cprow(r, cc2):
            rows[r, pl.ds(0, 16)] = r32[r, pl.ds(0, 16)]
            rows[r, pl.ds(16, 16)] = r32[r, pl.ds(16, 16)]
            return cc2

        lax.fori_loop(0, CH, cprow, 0)
        pltpu.sync_copy(rows, accS.at[sidxv], add=True)
        pltpu.sync_copy(rows, accT.at[tidxv], add=True)
        return cc

    lax.fori_loop(0, nch, chunk, 0)
    plsc.subcore_barrier()
    tsl = pl.ds(s * ROWS_PT, ROWS_PT)
    pltpu.sync_copy(accS.at[tsl], sp_hbm.at[c, tsl])
    pltpu.sync_copy(accT.at[tsl], tp_hbm.at[c, tsl])


def _scatter(x2, src, tgt):
    mesh = plsc.VectorSubcoreMesh(core_axis_name="c", subcore_axis_name="s")
    return pl.kernel(
        _scatter_body,
        out_type=(
            jax.ShapeDtypeStruct((NC, ACC_N, W), jnp.float32),
            jax.ShapeDtypeStruct((NC, ACC_N, W), jnp.float32),
        ),
        mesh=mesh,
        scratch_types=[
            pltpu.VMEM((CH, W), jnp.float32),
            pltpu.VMEM((CH, LATDIM), jnp.float32),
            pltpu.VMEM((CH,), jnp.int32),
            pltpu.VMEM((CH,), jnp.int32),
            pltpu.VMEM_SHARED((ACC_N, W), jnp.float32),
            pltpu.VMEM_SHARED((ACC_N, W), jnp.float32),
        ],
    )(x2, src, tgt)


def kernel(src, tgt, val, hyper, uEmbeds, iEmbeds, edgeTrans):
    uEp = jnp.zeros((TBL_N, LATDIM), jnp.float32).at[:USER].set(uEmbeds)
    iEp = jnp.zeros((TBL_N, LATDIM), jnp.float32).at[:ITEM].set(iEmbeds)
    uE4 = jnp.concatenate([uEp] * 4, axis=1)
    iE4 = jnp.concatenate([iEp] * 4, axis=1)
    srows, trows = _gather(uE4, iE4, src, tgt)
    val3 = val.reshape(NB, 1, BE)
    x2 = _dense(srows, trows, val3, edgeTrans, hyper)
    sp, tp = _scatter(x2, src, tgt)
    srcOut = (sp[0] + sp[1])[:USER, :LATDIM]
    tgtOut = (tp[0] + tp[1])[:ITEM, :LATDIM]
    return (srcOut, tgtOut)


# speedup vs baseline: 3.8937x; 1.0619x over previous
import jax
import jax.numpy as jnp
from jax import lax
from jax.experimental import pallas as pl
from jax.experimental.pallas import tpu as pltpu
from jax.experimental.pallas import tpu_sc as plsc

USER = 5000
ITEM = 5000
LATDIM = 32
ETYPE = 4
HYPERNUM = 64
E = 160000

NC = 2
NS = 16
NW = NC * NS
CH = 128
NCHUNKS = E // CH
NCH_BASE = NCHUNKS // NW
EXTRA = NCHUNKS % NW

TBL_N = 5120
STG = TBL_N // NS
W = 4 * LATDIM

ROWS_PT = 320
ACC_N = NS * ROWS_PT

BE = 1280
NB = E // BE


def _leaky(x):
    return jnp.where(x >= 0, x, 0.5 * x)


CH1 = 64
NCHUNKS1 = E // CH1
NCH1_BASE = NCHUNKS1 // NW
EXTRA1 = NCHUNKS1 % NW


def _gather_body(ue_hbm, ie_hbm, src_hbm, tgt_hbm, srows_hbm, trows_hbm,
                 usp, isp, sidxv, tidxv, urows, irows, u32, i32,
                 sem_u, sem_i, sem_o1, sem_o2):
    c = lax.axis_index("c")
    s = lax.axis_index("s")
    wid = s * NC + c
    nch = jnp.where(wid < EXTRA1, NCH1_BASE + 1, NCH1_BASE)

    for k in range(STG // CH1):
        off = s * STG + k * CH1
        pltpu.sync_copy(ue_hbm.at[pl.ds(off, CH1)], urows)
        pltpu.sync_copy(urows, usp.at[pl.ds(off, CH1)])
        pltpu.sync_copy(ie_hbm.at[pl.ds(off, CH1)], irows)
        pltpu.sync_copy(irows, isp.at[pl.ds(off, CH1)])
    plsc.subcore_barrier()

    def chunk(j, carry):
        base = (j * NW + wid) * CH1
        pltpu.sync_copy(src_hbm.at[pl.ds(base, CH1)], sidxv)
        pltpu.sync_copy(tgt_hbm.at[pl.ds(base, CH1)], tidxv)
        cp_u = pltpu.async_copy(usp.at[sidxv], urows, sem_u)
        cp_i = pltpu.async_copy(isp.at[tidxv], irows, sem_i)
        cp_u.wait()
        cp_i.wait()

        @pl.when(j > 0)
        def _():
            pbase = ((j - 1) * NW + wid) * CH1
            pltpu.make_async_copy(u32, srows_hbm.at[pl.ds(pbase, CH1)],
                                  sem_o1).wait()
            pltpu.make_async_copy(i32, trows_hbm.at[pl.ds(pbase, CH1)],
                                  sem_o2).wait()

        def cprow(r4, cc):
            for d in range(4):
                r = r4 * 4 + d
                u32[r, pl.ds(0, 16)] = urows[r, pl.ds(0, 16)]
                u32[r, pl.ds(16, 16)] = urows[r, pl.ds(16, 16)]
                i32[r, pl.ds(0, 16)] = irows[r, pl.ds(0, 16)]
                i32[r, pl.ds(16, 16)] = irows[r, pl.ds(16, 16)]
            return cc

        lax.fori_loop(0, CH1 // 4, cprow, 0)
        pltpu.async_copy(u32, srows_hbm.at[pl.ds(base, CH1)], sem_o1)
        pltpu.async_copy(i32, trows_hbm.at[pl.ds(base, CH1)], sem_o2)
        return carry

    lax.fori_loop(0, nch, chunk, 0)
    fbase = ((nch - 1) * NW + wid) * CH1
    pltpu.make_async_copy(u32, srows_hbm.at[pl.ds(fbase, CH1)], sem_o1).wait()
    pltpu.make_async_copy(i32, trows_hbm.at[pl.ds(fbase, CH1)], sem_o2).wait()


def _gather(uE4, iE4, src, tgt):
    mesh = plsc.VectorSubcoreMesh(core_axis_name="c", subcore_axis_name="s")
    return pl.kernel(
        _gather_body,
        out_type=(
            jax.ShapeDtypeStruct((E, LATDIM), jnp.float32),
            jax.ShapeDtypeStruct((E, LATDIM), jnp.float32),
        ),
        mesh=mesh,
        scratch_types=[
            pltpu.VMEM_SHARED((TBL_N, W), jnp.float32),
            pltpu.VMEM_SHARED((TBL_N, W), jnp.float32),
            pltpu.VMEM((CH1,), jnp.int32),
            pltpu.VMEM((CH1,), jnp.int32),
            pltpu.VMEM((CH1, W), jnp.float32),
            pltpu.VMEM((CH1, W), jnp.float32),
            pltpu.VMEM((CH1, LATDIM), jnp.float32),
            pltpu.VMEM((CH1, LATDIM), jnp.float32),
            pltpu.SemaphoreType.DMA,
            pltpu.SemaphoreType.DMA,
            pltpu.SemaphoreType.DMA,
            pltpu.SemaphoreType.DMA,
        ],
    )(uE4, iE4, src, tgt)


def _dense_body(s_ref, t_ref, v_ref, et_ref, h_ref, out_ref, Xs, A, P, xt):
    p = pl.program_id(0)
    i = pl.program_id(1)

    @pl.when(i == 0)
    def _():
        P[...] = _leaky(A[...])
        A[...] = jnp.zeros((LATDIM, HYPERNUM), jnp.float32)

    h = h_ref[...]

    @pl.when(p == 0)
    def _():
        sb = s_ref[...]
        vb = v_ref[0, 0, :].astype(jnp.int32)
        x0 = jnp.zeros((BE, LATDIM), jnp.float32)
        for v in range(ETYPE):
            y = jnp.dot(sb, et_ref[v], preferred_element_type=jnp.float32)
            mask = (vb == v).astype(jnp.float32)[:, None]
            x0 = x0 + mask * y
        x0 = x0 * t_ref[...]
        xt[...] = x0.T

    @pl.when(p > 0)
    def _():
        hp = lax.dot_general(P[...], h, (((1,), (1,)), ((), ())),
                             preferred_element_type=jnp.float32)
        xt[...] = _leaky(hp) + Xs[:, pl.ds(i * BE, BE)]

    x = xt[...]

    @pl.when(p < 2)
    def _():
        Xs[:, pl.ds(i * BE, BE)] = x
        A[...] += lax.dot_general(x, h, (((1,), (0,)), ((), ())),
                                  preferred_element_type=jnp.float32)

    @pl.when(p == 2)
    def _():
        out_ref[...] = x.T


def _dense(srows, trows, val3, edgeTrans, hyper):
    return pl.pallas_call(
        _dense_body,
        grid=(3, NB),
        in_specs=[
            pl.BlockSpec((BE, LATDIM), lambda p, i: (jnp.where(p == 0, i, 0), 0)),
            pl.BlockSpec((BE, LATDIM), lambda p, i: (jnp.where(p == 0, i, 0), 0)),
            pl.BlockSpec((1, 1, BE), lambda p, i: (jnp.where(p == 0, i, 0), 0, 0)),
            pl.BlockSpec((ETYPE, LATDIM, LATDIM), lambda p, i: (0, 0, 0)),
            pl.BlockSpec((BE, HYPERNUM), lambda p, i: (i, 0)),
        ],
        out_specs=pl.BlockSpec((BE, LATDIM), lambda p, i: (jnp.where(p == 2, i, 0), 0)),
        out_shape=jax.ShapeDtypeStruct((E, LATDIM), jnp.float32),
        scratch_shapes=[
            pltpu.VMEM((LATDIM, E), jnp.float32),
            pltpu.VMEM((LATDIM, HYPERNUM), jnp.float32),
            pltpu.VMEM((LATDIM, HYPERNUM), jnp.float32),
            pltpu.VMEM((LATDIM, BE), jnp.float32),
        ],
        compiler_params=pltpu.CompilerParams(
            dimension_semantics=("arbitrary", "arbitrary")),
    )(srows, trows, val3, edgeTrans, hyper)


def _scatter_body(x2_hbm, src_hbm, tgt_hbm, sp_hbm, tp_hbm,
                  rows, r32, sidxv, tidxv, accS, accT, sem_x):
    c = lax.axis_index("c")
    s = lax.axis_index("s")
    wid = s * NC + c
    nch = jnp.where(wid < EXTRA, NCH_BASE + 1, NCH_BASE)

    def zrow(r, cc):
        for q in range(W // 16):
            rows[r, pl.ds(q * 16, 16)] = jnp.zeros((16,), jnp.float32)
        return cc

    lax.fori_loop(0, CH, zrow, 0)
    for acc in (accS, accT):
        pltpu.sync_copy(rows, acc.at[pl.ds(s * ROWS_PT, CH)])
        pltpu.sync_copy(rows, acc.at[pl.ds(s * ROWS_PT + CH, CH)])
        pltpu.sync_copy(rows.at[pl.ds(0, ROWS_PT - 2 * CH)],
                        acc.at[pl.ds(s * ROWS_PT + 2 * CH, ROWS_PT - 2 * CH)])
    plsc.subcore_barrier()

    def chunk(j, cc):
        base = (j * NW + wid) * CH
        cp_x = pltpu.async_copy(x2_hbm.at[pl.ds(base, CH)], r32, sem_x)
        pltpu.sync_copy(src_hbm.at[pl.ds(base, CH)], sidxv)
        pltpu.sync_copy(tgt_hbm.at[pl.ds(base, CH)], tidxv)
        cp_x.wait()

        def cprow(r4, cc2):
            for d in range(4):
                r = r4 * 4 + d
                rows[r, pl.ds(0, 16)] = r32[r, pl.ds(0, 16)]
                rows[r, pl.ds(16, 16)] = r32[r, pl.ds(16, 16)]
            return cc2

        lax.fori_loop(0, CH // 4, cprow, 0)
        pltpu.sync_copy(rows, accS.at[sidxv], add=True)
        pltpu.sync_copy(rows, accT.at[tidxv], add=True)
        return cc

    lax.fori_loop(0, nch, chunk, 0)
    plsc.subcore_barrier()
    tsl = pl.ds(s * ROWS_PT, ROWS_PT)
    pltpu.sync_copy(accS.at[tsl], sp_hbm.at[c, tsl])
    pltpu.sync_copy(accT.at[tsl], tp_hbm.at[c, tsl])


def _scatter(x2, src, tgt):
    mesh = plsc.VectorSubcoreMesh(core_axis_name="c", subcore_axis_name="s")
    return pl.kernel(
        _scatter_body,
        out_type=(
            jax.ShapeDtypeStruct((NC, ACC_N, W), jnp.float32),
            jax.ShapeDtypeStruct((NC, ACC_N, W), jnp.float32),
        ),
        mesh=mesh,
        scratch_types=[
            pltpu.VMEM((CH, W), jnp.float32),
            pltpu.VMEM((CH, LATDIM), jnp.float32),
            pltpu.VMEM((CH,), jnp.int32),
            pltpu.VMEM((CH,), jnp.int32),
            pltpu.VMEM_SHARED((ACC_N, W), jnp.float32),
            pltpu.VMEM_SHARED((ACC_N, W), jnp.float32),
            pltpu.SemaphoreType.DMA,
        ],
    )(x2, src, tgt)


def kernel(src, tgt, val, hyper, uEmbeds, iEmbeds, edgeTrans):
    uEp = jnp.zeros((TBL_N, LATDIM), jnp.float32).at[:USER].set(uEmbeds)
    iEp = jnp.zeros((TBL_N, LATDIM), jnp.float32).at[:ITEM].set(iEmbeds)
    uE4 = jnp.concatenate([uEp] * 4, axis=1)
    iE4 = jnp.concatenate([iEp] * 4, axis=1)
    srows, trows = _gather(uE4, iE4, src, tgt)
    val3 = val.reshape(NB, 1, BE)
    x2 = _dense(srows, trows, val3, edgeTrans, hyper)
    sp, tp = _scatter(x2, src, tgt)
    srcOut = (sp[0] + sp[1])[:USER, :LATDIM]
    tgtOut = (tp[0] + tp[1])[:ITEM, :LATDIM]
    return (srcOut, tgtOut)
